# Initial kernel scaffold; baseline (speedup 1.0000x reference)
import jax
import jax.numpy as jnp
from jax.experimental import pallas as pl


def kernel(x, edge_index, edge_attr, Wi, bi, nW1, nb1, nW2, nb2, eW1, eb1, eW2, eb2, mW1, mb1, mW2, mb2, gamma, beta, Wo, bo):
    raise NotImplementedError("write your pallas kernel here")


# speedup vs baseline: 2.2814x; 2.2814x over previous
import functools

import jax
import jax.numpy as jnp
from jax import lax
from jax.experimental import pallas as pl
from jax.experimental.pallas import tpu as pltpu
from jax.experimental.pallas import tpu_sc as plsc

N = 10000
E = 320000
H = 128
L = 3

NC, NS = 2, 16
NW = NC * NS
EPW = E // NW
C = 80
NCHUNK = EPW // C
CPT = 1000
NTC = N // CPT

_BN_SCALE = float(1.0 / (1.0 + 1e-5) ** 0.5)


def _sc_mesh():
    return plsc.VectorSubcoreMesh(
        core_axis_name="c", subcore_axis_name="s", num_cores=NC, num_subcores=NS
    )


def _dense_body(x_ref, w_ref, b_ref, o_ref):
    o_ref[...] = (
        jnp.dot(x_ref[...], w_ref[...], preferred_element_type=jnp.float32)
        + b_ref[...]
    )


def _dense(x, w, b, bm=2000):
    n, k = x.shape
    m = w.shape[1]
    return pl.pallas_call(
        _dense_body,
        grid=(n // bm,),
        in_specs=[
            pl.BlockSpec((bm, k), lambda i: (i, 0)),
            pl.BlockSpec((k, m), lambda i: (0, 0)),
            pl.BlockSpec((1, m), lambda i: (0, 0)),
        ],
        out_specs=pl.BlockSpec((bm, m), lambda i: (i, 0)),
        out_shape=jax.ShapeDtypeStruct((n, m), jnp.float32),
    )(x, w, b.reshape(1, m))


def _prep_body(eW2_ref, mW1_ref, eb2_ref, mb1_ref, wc_ref, c_ref):
    mW1b = mW1_ref[0, H:, :]
    wc_ref[0] = jnp.dot(eW2_ref[0], mW1b, preferred_element_type=jnp.float32)
    c_ref[0] = (
        jnp.dot(eb2_ref[0], mW1b, preferred_element_type=jnp.float32)
        + mb1_ref[0]
    )


def _prep(eW2, mW1, eb2, mb1):
    wc, c = pl.pallas_call(
        _prep_body,
        grid=(L,),
        in_specs=[
            pl.BlockSpec((1, H, H), lambda i: (i, 0, 0)),
            pl.BlockSpec((1, 2 * H, H), lambda i: (i, 0, 0)),
            pl.BlockSpec((1, 1, H), lambda i: (i, 0, 0)),
            pl.BlockSpec((1, 1, H), lambda i: (i, 0, 0)),
        ],
        out_specs=[
            pl.BlockSpec((1, H, H), lambda i: (i, 0, 0)),
            pl.BlockSpec((1, 1, H), lambda i: (i, 0, 0)),
        ],
        out_shape=[
            jax.ShapeDtypeStruct((L, H, H), jnp.float32),
            jax.ShapeDtypeStruct((L, 1, H), jnp.float32),
        ],
    )(eW2, mW1, eb2.reshape(L, 1, H), mb1.reshape(L, 1, H))
    return wc, c.reshape(L, H)


def _node_body(h_ref, w1_ref, b1_ref, w2_ref, b2_ref, wa_ref, xn_ref, a_ref):
    t = jnp.maximum(
        jnp.dot(h_ref[...], w1_ref[...], preferred_element_type=jnp.float32)
        + b1_ref[...],
        0.0,
    )
    xn = (
        jnp.dot(t, w2_ref[...], preferred_element_type=jnp.float32) + b2_ref[...]
    )
    xn_ref[...] = xn
    a_ref[...] = jnp.dot(xn, wa_ref[...], preferred_element_type=jnp.float32)


def _node(h, w1, b1, w2, b2, wa, bm=2000):
    return pl.pallas_call(
        _node_body,
        grid=(N // bm,),
        in_specs=[
            pl.BlockSpec((bm, H), lambda i: (i, 0)),
            pl.BlockSpec((H, H), lambda i: (0, 0)),
            pl.BlockSpec((1, H), lambda i: (0, 0)),
            pl.BlockSpec((H, H), lambda i: (0, 0)),
            pl.BlockSpec((1, H), lambda i: (0, 0)),
            pl.BlockSpec((H, H), lambda i: (0, 0)),
        ],
        out_specs=[
            pl.BlockSpec((bm, H), lambda i: (i, 0)),
            pl.BlockSpec((bm, H), lambda i: (i, 0)),
        ],
        out_shape=[
            jax.ShapeDtypeStruct((N, H), jnp.float32),
            jax.ShapeDtypeStruct((N, H), jnp.float32),
        ],
    )(h, w1, b1.reshape(1, H), w2, b2.reshape(1, H), wa)


def _msg_body(g_ref, ea_ref, ew1_ref, eb1_ref, wc_ref, c_ref, r_ref):
    ea = ea_ref[...]
    u = (
        ea[:, 0:1] * ew1_ref[0:1, :]
        + ea[:, 1:2] * ew1_ref[1:2, :]
        + ea[:, 2:3] * ew1_ref[2:3, :]
        + eb1_ref[...]
    )
    u = jnp.maximum(u, 0.0)
    v = jnp.dot(u, wc_ref[...], preferred_element_type=jnp.float32) + c_ref[...]
    r_ref[...] = jnp.maximum(g_ref[...] + v, 0.0)


def _msg(g, ea, ew1, eb1, wc, c, bm=4000):
    return pl.pallas_call(
        _msg_body,
        grid=(E // bm,),
        in_specs=[
            pl.BlockSpec((bm, H), lambda i: (i, 0)),
            pl.BlockSpec((bm, 3), lambda i: (i, 0)),
            pl.BlockSpec((3, H), lambda i: (0, 0)),
            pl.BlockSpec((1, H), lambda i: (0, 0)),
            pl.BlockSpec((H, H), lambda i: (0, 0)),
            pl.BlockSpec((1, H), lambda i: (0, 0)),
        ],
        out_specs=pl.BlockSpec((bm, H), lambda i: (i, 0)),
        out_shape=jax.ShapeDtypeStruct((E, H), jnp.float32),
    )(g, ea, ew1, eb1.reshape(1, H), wc, c.reshape(1, H))


def _post_body(
    s0_ref, s1_ref, c0_ref, c1_ref, xn_ref, h_ref, w2_ref, b2_ref, gb_ref, o_ref
):
    cnt = c0_ref[...] + c1_ref[...]
    s = s0_ref[...] + s1_ref[...]
    mean = s / jnp.maximum(cnt, 1.0)
    agg = jnp.dot(mean, w2_ref[...], preferred_element_type=jnp.float32) + b2_ref[...]
    agg = jnp.where(cnt > 0.0, agg, 0.0)
    xnew = agg + xn_ref[...]
    xnew = gb_ref[0:1, :] * xnew * _BN_SCALE + gb_ref[1:2, :]
    o_ref[...] = h_ref[...] + jnp.maximum(xnew, 0.0)


def _post(s0, s1, c0, c1, xn, h, w2, b2, gamma, beta, bm=2000):
    gb = jnp.stack([gamma, beta], axis=0)
    return pl.pallas_call(
        _post_body,
        grid=(N // bm,),
        in_specs=[
            pl.BlockSpec((bm, H), lambda i: (i, 0)),
            pl.BlockSpec((bm, H), lambda i: (i, 0)),
            pl.BlockSpec((bm, 1), lambda i: (i, 0)),
            pl.BlockSpec((bm, 1), lambda i: (i, 0)),
            pl.BlockSpec((bm, H), lambda i: (i, 0)),
            pl.BlockSpec((bm, H), lambda i: (i, 0)),
            pl.BlockSpec((H, H), lambda i: (0, 0)),
            pl.BlockSpec((1, H), lambda i: (0, 0)),
            pl.BlockSpec((2, H), lambda i: (0, 0)),
        ],
        out_specs=pl.BlockSpec((bm, H), lambda i: (i, 0)),
        out_shape=jax.ShapeDtypeStruct((N, H), jnp.float32),
    )(s0, s1, c0, c1, xn, h, w2, b2.reshape(1, H), gb)


def _gather(a, src):
    @functools.partial(
        pl.kernel,
        out_type=jax.ShapeDtypeStruct((E, H), jnp.float32),
        mesh=_sc_mesh(),
        scratch_types=[
            pltpu.VMEM((C,), jnp.int32),
            pltpu.VMEM((C, H), jnp.float32),
            pltpu.SemaphoreType.DMA,
        ],
    )
    def k(a_hbm, src_hbm, g_hbm, idx_v, rows_v, sem):
        wid = lax.axis_index("s") * NC + lax.axis_index("c")

        def body(j, carry):
            base = wid * EPW + j * C
            pltpu.sync_copy(src_hbm.at[pl.ds(base, C)], idx_v)
            pltpu.async_copy(a_hbm.at[idx_v], rows_v, sem).wait()
            pltpu.sync_copy(rows_v, g_hbm.at[pl.ds(base, C)])
            return carry

        lax.fori_loop(0, NCHUNK, body, 0)

    return k(a, src)


def _scatter(r, dst, zrows):
    @functools.partial(
        pl.kernel,
        out_type=jax.ShapeDtypeStruct((NC, N, H), jnp.float32),
        mesh=_sc_mesh(),
        scratch_types=[
            pltpu.VMEM((C,), jnp.int32),
            pltpu.VMEM((C, H), jnp.float32),
            pltpu.VMEM_SHARED((N, H), jnp.float32),
        ],
    )
    def k(r_hbm, dst_hbm, z_hbm, out_hbm, idx_v, rows_v, s_sh):
        cid = lax.axis_index("c")
        sid = lax.axis_index("s")
        wid = sid * NC + cid

        @pl.when(sid < NTC)
        def _zero():
            pltpu.sync_copy(z_hbm, s_sh.at[pl.ds(sid * CPT, CPT)])

        plsc.subcore_barrier()

        def body(j, carry):
            base = wid * EPW + j * C
            pltpu.sync_copy(dst_hbm.at[pl.ds(base, C)], idx_v)
            pltpu.sync_copy(r_hbm.at[pl.ds(base, C)], rows_v)
            pltpu.sync_copy(rows_v, s_sh.at[idx_v], add=True)
            return carry

        lax.fori_loop(0, NCHUNK, body, 0)
        plsc.subcore_barrier()

        @pl.when(sid < NTC)
        def _out():
            pltpu.sync_copy(
                s_sh.at[pl.ds(sid * CPT, CPT)],
                out_hbm.at[cid].at[pl.ds(sid * CPT, CPT)],
            )

    return k(r, dst, zrows)


def _degree(dst, ones_rows, zrows):
    @functools.partial(
        pl.kernel,
        out_type=jax.ShapeDtypeStruct((NC, N, H), jnp.float32),
        mesh=_sc_mesh(),
        scratch_types=[
            pltpu.VMEM((C,), jnp.int32),
            pltpu.VMEM((C, H), jnp.float32),
            pltpu.VMEM_SHARED((N, H), jnp.float32),
        ],
    )
    def k(dst_hbm, ones_hbm, z_hbm, out_hbm, idx_v, ones_v, cnt_sh):
        cid = lax.axis_index("c")
        sid = lax.axis_index("s")
        wid = sid * NC + cid
        pltpu.sync_copy(ones_hbm, ones_v)

        @pl.when(sid < NTC)
        def _zero():
            pltpu.sync_copy(z_hbm, cnt_sh.at[pl.ds(sid * CPT, CPT)])

        plsc.subcore_barrier()

        def body(j, carry):
            base = wid * EPW + j * C
            pltpu.sync_copy(dst_hbm.at[pl.ds(base, C)], idx_v)
            pltpu.sync_copy(ones_v, cnt_sh.at[idx_v], add=True)
            return carry

        lax.fori_loop(0, NCHUNK, body, 0)
        plsc.subcore_barrier()

        @pl.when(sid < NTC)
        def _out():
            pltpu.sync_copy(
                cnt_sh.at[pl.ds(sid * CPT, CPT)],
                out_hbm.at[cid].at[pl.ds(sid * CPT, CPT)],
            )

    return k(dst, ones_rows, zrows)


def kernel(x, edge_index, edge_attr, Wi, bi, nW1, nb1, nW2, nb2, eW1, eb1,
           eW2, eb2, mW1, mb1, mW2, mb2, gamma, beta, Wo, bo):
    src = edge_index[0]
    dst = edge_index[1]

    wc_all, c_all = _prep(eW2, mW1, eb2, mb1)

    zrows = jnp.zeros((CPT, H), jnp.float32)
    ones_rows = jnp.ones((C, H), jnp.float32)

    deg = _degree(dst, ones_rows, zrows)
    c0 = deg[0, :, 0:1]
    c1 = deg[1, :, 0:1]

    h = _dense(x, Wi, bi)
    for i in range(L):
        xn, a = _node(h, nW1[i], nb1[i], nW2[i], nb2[i], mW1[i, :H, :])
        g = _gather(a, src)
        r = _msg(g, edge_attr, eW1[i], eb1[i], wc_all[i], c_all[i])
        s = _scatter(r, dst, zrows)
        h = _post(s[0], s[1], c0, c1, xn, h, mW2[i], mb2[i], gamma[i], beta[i])
    return _dense(h, Wo, bo)


# speedup vs baseline: 3.4692x; 1.5207x over previous
import functools

import jax
import jax.numpy as jnp
from jax import lax
from jax.experimental import pallas as pl
from jax.experimental.pallas import tpu as pltpu
from jax.experimental.pallas import tpu_sc as plsc

N = 10000
E = 320000
H = 128
L = 3

NC, NS = 2, 16
NW = NC * NS
EPW = E // NW
CG = 80
NCHG = EPW // CG
CS = 40
NCHS = EPW // CS
CPT = 1000
NTC = N // CPT

_BN_SCALE = float(1.0 / (1.0 + 1e-5) ** 0.5)


def _sc_mesh():
    return plsc.VectorSubcoreMesh(
        core_axis_name="c", subcore_axis_name="s", num_cores=NC, num_subcores=NS
    )


def _dense_body(x_ref, w_ref, b_ref, o_ref):
    o_ref[...] = (
        jnp.dot(x_ref[...], w_ref[...], preferred_element_type=jnp.float32)
        + b_ref[...]
    )


def _dense(x, w, b, bm=2000):
    n, k = x.shape
    m = w.shape[1]
    return pl.pallas_call(
        _dense_body,
        grid=(n // bm,),
        in_specs=[
            pl.BlockSpec((bm, k), lambda i: (i, 0)),
            pl.BlockSpec((k, m), lambda i: (0, 0)),
            pl.BlockSpec((1, m), lambda i: (0, 0)),
        ],
        out_specs=pl.BlockSpec((bm, m), lambda i: (i, 0)),
        out_shape=jax.ShapeDtypeStruct((n, m), jnp.float32),
    )(x, w, b.reshape(1, m))


def _prep_body(eW2_ref, mW1_ref, eb2_ref, mb1_ref, wc_ref, c_ref):
    mW1b = mW1_ref[0, H:, :]
    wc_ref[0] = jnp.dot(eW2_ref[0], mW1b, preferred_element_type=jnp.float32)
    c_ref[0] = (
        jnp.dot(eb2_ref[0], mW1b, preferred_element_type=jnp.float32)
        + mb1_ref[0]
    )


def _prep(eW2, mW1, eb2, mb1):
    wc, c = pl.pallas_call(
        _prep_body,
        grid=(L,),
        in_specs=[
            pl.BlockSpec((1, H, H), lambda i: (i, 0, 0)),
            pl.BlockSpec((1, 2 * H, H), lambda i: (i, 0, 0)),
            pl.BlockSpec((1, 1, H), lambda i: (i, 0, 0)),
            pl.BlockSpec((1, 1, H), lambda i: (i, 0, 0)),
        ],
        out_specs=[
            pl.BlockSpec((1, H, H), lambda i: (i, 0, 0)),
            pl.BlockSpec((1, 1, H), lambda i: (i, 0, 0)),
        ],
        out_shape=[
            jax.ShapeDtypeStruct((L, H, H), jnp.float32),
            jax.ShapeDtypeStruct((L, 1, H), jnp.float32),
        ],
    )(eW2, mW1, eb2.reshape(L, 1, H), mb1.reshape(L, 1, H))
    return wc, c.reshape(L, H)


def _node_body(h_ref, w1_ref, b1_ref, w2_ref, b2_ref, wa_ref, xn_ref, a_ref):
    t = jnp.maximum(
        jnp.dot(h_ref[...], w1_ref[...], preferred_element_type=jnp.float32)
        + b1_ref[...],
        0.0,
    )
    xn = (
        jnp.dot(t, w2_ref[...], preferred_element_type=jnp.float32) + b2_ref[...]
    )
    xn_ref[...] = xn
    a_ref[...] = jnp.dot(xn, wa_ref[...], preferred_element_type=jnp.float32)


def _node(h, w1, b1, w2, b2, wa, bm=2000):
    return pl.pallas_call(
        _node_body,
        grid=(N // bm,),
        in_specs=[
            pl.BlockSpec((bm, H), lambda i: (i, 0)),
            pl.BlockSpec((H, H), lambda i: (0, 0)),
            pl.BlockSpec((1, H), lambda i: (0, 0)),
            pl.BlockSpec((H, H), lambda i: (0, 0)),
            pl.BlockSpec((1, H), lambda i: (0, 0)),
            pl.BlockSpec((H, H), lambda i: (0, 0)),
        ],
        out_specs=[
            pl.BlockSpec((bm, H), lambda i: (i, 0)),
            pl.BlockSpec((bm, H), lambda i: (i, 0)),
        ],
        out_shape=[
            jax.ShapeDtypeStruct((N, H), jnp.float32),
            jax.ShapeDtypeStruct((N, H), jnp.float32),
        ],
    )(h, w1, b1.reshape(1, H), w2, b2.reshape(1, H), wa)


def _msg_body(g_ref, ea_ref, ew1_ref, eb1_ref, wc_ref, c_ref, r_ref):
    ea = ea_ref[...]
    u = (
        ea[:, 0:1] * ew1_ref[0:1, :]
        + ea[:, 1:2] * ew1_ref[1:2, :]
        + ea[:, 2:3] * ew1_ref[2:3, :]
        + eb1_ref[...]
    )
    u = jnp.maximum(u, 0.0)
    v = jnp.dot(u, wc_ref[...], preferred_element_type=jnp.float32) + c_ref[...]
    r_ref[...] = jnp.maximum(g_ref[...] + v, 0.0)


def _msg(g, ea, ew1, eb1, wc, c, bm=4000):
    return pl.pallas_call(
        _msg_body,
        grid=(E // bm,),
        in_specs=[
            pl.BlockSpec((bm, H), lambda i: (i, 0)),
            pl.BlockSpec((bm, 3), lambda i: (i, 0)),
            pl.BlockSpec((3, H), lambda i: (0, 0)),
            pl.BlockSpec((1, H), lambda i: (0, 0)),
            pl.BlockSpec((H, H), lambda i: (0, 0)),
            pl.BlockSpec((1, H), lambda i: (0, 0)),
        ],
        out_specs=pl.BlockSpec((bm, H), lambda i: (i, 0)),
        out_shape=jax.ShapeDtypeStruct((E, H), jnp.float32),
    )(g, ea, ew1, eb1.reshape(1, H), wc, c.reshape(1, H))


def _post_body(
    s0_ref, s1_ref, c0_ref, c1_ref, xn_ref, h_ref, w2_ref, b2_ref, gb_ref, o_ref
):
    cnt = c0_ref[...] + c1_ref[...]
    s = s0_ref[...] + s1_ref[...]
    mean = s / jnp.maximum(cnt, 1.0)
    agg = jnp.dot(mean, w2_ref[...], preferred_element_type=jnp.float32) + b2_ref[...]
    agg = jnp.where(cnt > 0.0, agg, 0.0)
    xnew = agg + xn_ref[...]
    xnew = gb_ref[0:1, :] * xnew * _BN_SCALE + gb_ref[1:2, :]
    o_ref[...] = h_ref[...] + jnp.maximum(xnew, 0.0)


def _post(s0, s1, c0, c1, xn, h, w2, b2, gamma, beta, bm=2000):
    gb = jnp.stack([gamma, beta], axis=0)
    return pl.pallas_call(
        _post_body,
        grid=(N // bm,),
        in_specs=[
            pl.BlockSpec((bm, H), lambda i: (i, 0)),
            pl.BlockSpec((bm, H), lambda i: (i, 0)),
            pl.BlockSpec((bm, 1), lambda i: (i, 0)),
            pl.BlockSpec((bm, 1), lambda i: (i, 0)),
            pl.BlockSpec((bm, H), lambda i: (i, 0)),
            pl.BlockSpec((bm, H), lambda i: (i, 0)),
            pl.BlockSpec((H, H), lambda i: (0, 0)),
            pl.BlockSpec((1, H), lambda i: (0, 0)),
            pl.BlockSpec((2, H), lambda i: (0, 0)),
        ],
        out_specs=pl.BlockSpec((bm, H), lambda i: (i, 0)),
        out_shape=jax.ShapeDtypeStruct((N, H), jnp.float32),
    )(s0, s1, c0, c1, xn, h, w2, b2.reshape(1, H), gb)


DEPTH = 8
NBUF = 5


def _gather(a, src2):
    @functools.partial(
        pl.kernel,
        out_type=jax.ShapeDtypeStruct((E, H), jnp.float32),
        mesh=_sc_mesh(),
        scratch_types=[
            pltpu.VMEM((NCHG, CG), jnp.int32),
            pltpu.VMEM((NBUF, CG, H), jnp.float32),
            pltpu.SemaphoreType.DMA,
            pltpu.SemaphoreType.DMA,
        ],
    )
    def k(a_hbm, src_hbm, g_hbm, idx2d, bufs, gsem, wsem):
        wid = lax.axis_index("s") * NC + lax.axis_index("c")
        pltpu.sync_copy(src_hbm.at[wid], idx2d)

        def body(t, carry):
            @pl.when(t > 0)
            def _drain_w():
                for b in range(NBUF):
                    pltpu.make_async_copy(
                        bufs.at[b], g_hbm.at[pl.ds(wid * EPW, CG)], wsem
                    ).wait()

            for b in range(NBUF):
                pltpu.async_copy(
                    a_hbm.at[idx2d.at[t * NBUF + b]], bufs.at[b], gsem
                )
            for b in range(NBUF):
                pltpu.make_async_copy(
                    a_hbm.at[idx2d.at[0]], bufs.at[b], gsem
                ).wait()
            for b in range(NBUF):
                base = wid * EPW + (t * NBUF + b) * CG
                pltpu.async_copy(bufs.at[b], g_hbm.at[pl.ds(base, CG)], wsem)
            return carry

        lax.fori_loop(0, NCHG // NBUF, body, 0)
        for b in range(NBUF):
            pltpu.make_async_copy(
                bufs.at[b], g_hbm.at[pl.ds(wid * EPW, CG)], wsem
            ).wait()

    return k(a, src2)


def _scatter(r, dst2, zrows):
    @functools.partial(
        pl.kernel,
        out_type=jax.ShapeDtypeStruct((NC, N, H), jnp.float32),
        mesh=_sc_mesh(),
        scratch_types=[
            pltpu.VMEM((NBUF, CS), jnp.int32),
            pltpu.VMEM((NBUF, CS, H), jnp.float32),
            pltpu.VMEM_SHARED((N, H), jnp.float32),
            [pltpu.SemaphoreType.DMA] * NBUF,
            [pltpu.SemaphoreType.DMA] * NBUF,
        ],
    )
    def k(r_hbm, dst_hbm, z_hbm, out_hbm, idxb, bufs, s_sh, lsems, asems):
        cid = lax.axis_index("c")
        sid = lax.axis_index("s")
        wid = sid * NC + cid

        @pl.when(sid < NTC)
        def _zero():
            pltpu.sync_copy(z_hbm, s_sh.at[pl.ds(sid * CPT, CPT)])

        plsc.subcore_barrier()

        def body(t, carry):
            for b in range(NBUF):
                j = t * NBUF + b

                @pl.when(t > 0)
                def _drain_a(b=b):
                    pltpu.make_async_copy(
                        bufs.at[b], s_sh.at[idxb.at[b]], asems[b]
                    ).wait()

                pltpu.async_copy(dst_hbm.at[wid].at[j], idxb.at[b], lsems[b])
                base = wid * EPW + j * CS
                pltpu.async_copy(r_hbm.at[pl.ds(base, CS)], bufs.at[b], lsems[b])
            for b in range(NBUF):
                j = t * NBUF + b
                pltpu.make_async_copy(
                    dst_hbm.at[wid].at[0], idxb.at[b], lsems[b]
                ).wait()
                pltpu.make_async_copy(
                    r_hbm.at[pl.ds(wid * EPW, CS)], bufs.at[b], lsems[b]
                ).wait()
                pltpu.async_copy(
                    bufs.at[b], s_sh.at[idxb.at[b]], asems[b], add=True
                )
            return carry

        lax.fori_loop(0, NCHS // NBUF, body, 0)
        for b in range(NBUF):
            pltpu.make_async_copy(bufs.at[b], s_sh.at[idxb.at[b]], asems[b]).wait()
        plsc.subcore_barrier()

        @pl.when(sid < NTC)
        def _out():
            pltpu.sync_copy(
                s_sh.at[pl.ds(sid * CPT, CPT)],
                out_hbm.at[cid].at[pl.ds(sid * CPT, CPT)],
            )

    return k(r, dst2, zrows)


def _degree(dst2, ones_rows, zrows):
    @functools.partial(
        pl.kernel,
        out_type=jax.ShapeDtypeStruct((NC, N, H), jnp.float32),
        mesh=_sc_mesh(),
        scratch_types=[
            pltpu.VMEM((NCHS, CS), jnp.int32),
            pltpu.VMEM((CS, H), jnp.float32),
            pltpu.VMEM_SHARED((N, H), jnp.float32),
            pltpu.SemaphoreType.DMA,
        ],
    )
    def k(dst_hbm, ones_hbm, z_hbm, out_hbm, idx2d, ones_v, cnt_sh, sem):
        cid = lax.axis_index("c")
        sid = lax.axis_index("s")
        wid = sid * NC + cid
        pltpu.sync_copy(dst_hbm.at[wid], idx2d)
        pltpu.sync_copy(ones_hbm, ones_v)

        @pl.when(sid < NTC)
        def _zero():
            pltpu.sync_copy(z_hbm, cnt_sh.at[pl.ds(sid * CPT, CPT)])

        plsc.subcore_barrier()

        def body(j, carry):
            d = pltpu.async_copy(ones_v, cnt_sh.at[idx2d.at[j]], sem, add=True)

            @pl.when(j >= DEPTH)
            def _drain():
                d.wait()

            return carry

        lax.fori_loop(0, NCHS, body, 0)
        for _ in range(DEPTH):
            pltpu.make_async_copy(ones_v, cnt_sh.at[idx2d.at[0]], sem).wait()
        plsc.subcore_barrier()

        @pl.when(sid < NTC)
        def _out():
            pltpu.sync_copy(
                cnt_sh.at[pl.ds(sid * CPT, CPT)],
                out_hbm.at[cid].at[pl.ds(sid * CPT, CPT)],
            )

    return k(dst2, ones_rows, zrows)


def kernel(x, edge_index, edge_attr, Wi, bi, nW1, nb1, nW2, nb2, eW1, eb1,
           eW2, eb2, mW1, mb1, mW2, mb2, gamma, beta, Wo, bo):
    src2 = edge_index[0].reshape(NW, NCHG, CG)
    dst2 = edge_index[1].reshape(NW, NCHS, CS)

    wc_all, c_all = _prep(eW2, mW1, eb2, mb1)

    zrows = jnp.zeros((CPT, H), jnp.float32)
    ones_rows = jnp.ones((CS, H), jnp.float32)

    deg = _degree(dst2, ones_rows, zrows)
    c0 = deg[0, :, 0:1]
    c1 = deg[1, :, 0:1]

    h = _dense(x, Wi, bi)
    for i in range(L):
        xn, a = _node(h, nW1[i], nb1[i], nW2[i], nb2[i], mW1[i, :H, :])
        g = _gather(a, src2)
        r = _msg(g, edge_attr, eW1[i], eb1[i], wc_all[i], c_all[i])
        s = _scatter(r, dst2, zrows)
        h = _post(s[0], s[1], c0, c1, xn, h, mW2[i], mb2[i], gamma[i], beta[i])
    return _dense(h, Wo, bo)


# speedup vs baseline: 3.5898x; 1.0348x over previous
import functools

import jax
import jax.numpy as jnp
from jax import lax
from jax.experimental import pallas as pl
from jax.experimental.pallas import tpu as pltpu
from jax.experimental.pallas import tpu_sc as plsc

N = 10000
E = 320000
H = 128
L = 3
IN_DIM = 128
OUT_DIM = 128

NC, NS = 2, 16
NW = NC * NS
EPW = E // NW
CG = 80
NCHG = EPW // CG
CS = 40
NCHS = EPW // CS
CPT = 1000
NTC = N // CPT

_BN_SCALE = float(1.0 / (1.0 + 1e-5) ** 0.5)


def _sc_mesh():
    return plsc.VectorSubcoreMesh(
        core_axis_name="c", subcore_axis_name="s", num_cores=NC, num_subcores=NS
    )


def _prep_body(eW2_ref, mW1_ref, eb2_ref, mb1_ref, wc_ref, c_ref):
    mW1b = mW1_ref[0, H:, :]
    wc_ref[0] = jnp.dot(eW2_ref[0], mW1b, preferred_element_type=jnp.float32)
    c_ref[0] = (
        jnp.dot(eb2_ref[0], mW1b, preferred_element_type=jnp.float32)
        + mb1_ref[0]
    )


def _prep(eW2, mW1, eb2, mb1):
    wc, c = pl.pallas_call(
        _prep_body,
        grid=(L,),
        in_specs=[
            pl.BlockSpec((1, H, H), lambda i: (i, 0, 0)),
            pl.BlockSpec((1, 2 * H, H), lambda i: (i, 0, 0)),
            pl.BlockSpec((1, 1, H), lambda i: (i, 0, 0)),
            pl.BlockSpec((1, 1, H), lambda i: (i, 0, 0)),
        ],
        out_specs=[
            pl.BlockSpec((1, H, H), lambda i: (i, 0, 0)),
            pl.BlockSpec((1, 1, H), lambda i: (i, 0, 0)),
        ],
        out_shape=[
            jax.ShapeDtypeStruct((L, H, H), jnp.float32),
            jax.ShapeDtypeStruct((L, 1, H), jnp.float32),
        ],
    )(eW2, mW1, eb2.reshape(L, 1, H), mb1.reshape(L, 1, H))
    return wc, c.reshape(L, H)


def _node_mlp(h, w1_ref, b1_ref, w2_ref, b2_ref, wa_ref):
    t = jnp.maximum(
        jnp.dot(h, w1_ref[...], preferred_element_type=jnp.float32)
        + b1_ref[...],
        0.0,
    )
    xn = jnp.dot(t, w2_ref[...], preferred_element_type=jnp.float32) + b2_ref[...]
    a = jnp.dot(xn, wa_ref[...], preferred_element_type=jnp.float32)
    return xn, a


def _in_node_body(x_ref, wi_ref, bi_ref, w1_ref, b1_ref, w2_ref, b2_ref,
                  wa_ref, h_ref, xn_ref, a_ref):
    h = (
        jnp.dot(x_ref[...], wi_ref[...], preferred_element_type=jnp.float32)
        + bi_ref[...]
    )
    h_ref[...] = h
    xn, a = _node_mlp(h, w1_ref, b1_ref, w2_ref, b2_ref, wa_ref)
    xn_ref[...] = xn
    a_ref[...] = a


def _in_node(x, Wi, bi, w1, b1, w2, b2, wa, bm=2000):
    wspec = pl.BlockSpec((H, H), lambda i: (0, 0))
    bspec = pl.BlockSpec((1, H), lambda i: (0, 0))
    rspec = pl.BlockSpec((bm, H), lambda i: (i, 0))
    return pl.pallas_call(
        _in_node_body,
        grid=(N // bm,),
        in_specs=[
            pl.BlockSpec((bm, IN_DIM), lambda i: (i, 0)),
            pl.BlockSpec((IN_DIM, H), lambda i: (0, 0)),
            bspec, wspec, bspec, wspec, bspec, wspec,
        ],
        out_specs=[rspec, rspec, rspec],
        out_shape=[
            jax.ShapeDtypeStruct((N, H), jnp.float32),
            jax.ShapeDtypeStruct((N, H), jnp.float32),
            jax.ShapeDtypeStruct((N, H), jnp.float32),
        ],
    )(x, Wi, bi.reshape(1, H), w1, b1.reshape(1, H), w2, b2.reshape(1, H), wa)


def _post_update(s0_ref, s1_ref, c0_ref, c1_ref, xn_ref, h_ref, w2_ref,
                 b2_ref, gb_ref):
    cnt = c0_ref[...] + c1_ref[...]
    s = s0_ref[...] + s1_ref[...]
    mean = s / jnp.maximum(cnt, 1.0)
    agg = jnp.dot(mean, w2_ref[...], preferred_element_type=jnp.float32) + b2_ref[...]
    agg = jnp.where(cnt > 0.0, agg, 0.0)
    xnew = agg + xn_ref[...]
    xnew = gb_ref[0:1, :] * xnew * _BN_SCALE + gb_ref[1:2, :]
    return h_ref[...] + jnp.maximum(xnew, 0.0)


def _post_node_body(s0_ref, s1_ref, c0_ref, c1_ref, xn_ref, h_ref, w2_ref,
                    b2_ref, gb_ref, nw1_ref, nb1_ref, nw2_ref, nb2_ref,
                    wa_ref, h_out_ref, xn_out_ref, a_out_ref):
    h = _post_update(s0_ref, s1_ref, c0_ref, c1_ref, xn_ref, h_ref, w2_ref,
                     b2_ref, gb_ref)
    h_out_ref[...] = h
    xn, a = _node_mlp(h, nw1_ref, nb1_ref, nw2_ref, nb2_ref, wa_ref)
    xn_out_ref[...] = xn
    a_out_ref[...] = a


def _post_node(s0, s1, c0, c1, xn, h, w2, b2, gamma, beta,
               nw1, nb1, nw2, nb2, wa, bm=2000):
    gb = jnp.stack([gamma, beta], axis=0)
    wspec = pl.BlockSpec((H, H), lambda i: (0, 0))
    bspec = pl.BlockSpec((1, H), lambda i: (0, 0))
    rspec = pl.BlockSpec((bm, H), lambda i: (i, 0))
    cspec = pl.BlockSpec((bm, 1), lambda i: (i, 0))
    return pl.pallas_call(
        _post_node_body,
        grid=(N // bm,),
        in_specs=[
            rspec, rspec, cspec, cspec, rspec, rspec,
            wspec, bspec, pl.BlockSpec((2, H), lambda i: (0, 0)),
            wspec, bspec, wspec, bspec, wspec,
        ],
        out_specs=[rspec, rspec, rspec],
        out_shape=[
            jax.ShapeDtypeStruct((N, H), jnp.float32),
            jax.ShapeDtypeStruct((N, H), jnp.float32),
            jax.ShapeDtypeStruct((N, H), jnp.float32),
        ],
    )(s0, s1, c0, c1, xn, h, w2, b2.reshape(1, H), gb,
      nw1, nb1.reshape(1, H), nw2, nb2.reshape(1, H), wa)


def _post_out_body(s0_ref, s1_ref, c0_ref, c1_ref, xn_ref, h_ref, w2_ref,
                   b2_ref, gb_ref, wo_ref, bo_ref, o_ref):
    h = _post_update(s0_ref, s1_ref, c0_ref, c1_ref, xn_ref, h_ref, w2_ref,
                     b2_ref, gb_ref)
    o_ref[...] = (
        jnp.dot(h, wo_ref[...], preferred_element_type=jnp.float32)
        + bo_ref[...]
    )


def _post_out(s0, s1, c0, c1, xn, h, w2, b2, gamma, beta, Wo, bo, bm=2000):
    gb = jnp.stack([gamma, beta], axis=0)
    bspec = pl.BlockSpec((1, H), lambda i: (0, 0))
    rspec = pl.BlockSpec((bm, H), lambda i: (i, 0))
    cspec = pl.BlockSpec((bm, 1), lambda i: (i, 0))
    return pl.pallas_call(
        _post_out_body,
        grid=(N // bm,),
        in_specs=[
            rspec, rspec, cspec, cspec, rspec, rspec,
            pl.BlockSpec((H, H), lambda i: (0, 0)), bspec,
            pl.BlockSpec((2, H), lambda i: (0, 0)),
            pl.BlockSpec((H, OUT_DIM), lambda i: (0, 0)),
            pl.BlockSpec((1, OUT_DIM), lambda i: (0, 0)),
        ],
        out_specs=pl.BlockSpec((bm, OUT_DIM), lambda i: (i, 0)),
        out_shape=jax.ShapeDtypeStruct((N, OUT_DIM), jnp.float32),
    )(s0, s1, c0, c1, xn, h, w2, b2.reshape(1, H), gb, Wo, bo.reshape(1, OUT_DIM))


def _msg_body(g_ref, ea_ref, ew1_ref, eb1_ref, wc_ref, c_ref, r_ref):
    ea = ea_ref[...]
    u = (
        ea[:, 0:1] * ew1_ref[0:1, :]
        + ea[:, 1:2] * ew1_ref[1:2, :]
        + ea[:, 2:3] * ew1_ref[2:3, :]
        + eb1_ref[...]
    )
    u = jnp.maximum(u, 0.0)
    v = jnp.dot(u, wc_ref[...], preferred_element_type=jnp.float32) + c_ref[...]
    r_ref[...] = jnp.maximum(g_ref[...] + v, 0.0)


def _msg(g, ea, ew1, eb1, wc, c, bm=4000):
    return pl.pallas_call(
        _msg_body,
        grid=(E // bm,),
        in_specs=[
            pl.BlockSpec((bm, H), lambda i: (i, 0)),
            pl.BlockSpec((bm, 3), lambda i: (i, 0)),
            pl.BlockSpec((3, H), lambda i: (0, 0)),
            pl.BlockSpec((1, H), lambda i: (0, 0)),
            pl.BlockSpec((H, H), lambda i: (0, 0)),
            pl.BlockSpec((1, H), lambda i: (0, 0)),
        ],
        out_specs=pl.BlockSpec((bm, H), lambda i: (i, 0)),
        out_shape=jax.ShapeDtypeStruct((E, H), jnp.float32),
    )(g, ea, ew1, eb1.reshape(1, H), wc, c.reshape(1, H))


DEPTH = 8
NBUF = 5


def _gather(a, src2):
    @functools.partial(
        pl.kernel,
        out_type=jax.ShapeDtypeStruct((E, H), jnp.float32),
        mesh=_sc_mesh(),
        scratch_types=[
            pltpu.VMEM((NCHG, CG), jnp.int32),
            pltpu.VMEM((NBUF, CG, H), jnp.float32),
            [pltpu.SemaphoreType.DMA] * NBUF,
            [pltpu.SemaphoreType.DMA] * NBUF,
        ],
    )
    def k(a_hbm, src_hbm, g_hbm, idx2d, bufs, gsems, wsems):
        wid = lax.axis_index("s") * NC + lax.axis_index("c")
        pltpu.sync_copy(src_hbm.at[wid], idx2d)

        def body(t, carry):
            for b in range(NBUF):
                @pl.when(t > 0)
                def _drain_w(b=b):
                    pltpu.make_async_copy(
                        bufs.at[b], g_hbm.at[pl.ds(wid * EPW, CG)], wsems[b]
                    ).wait()

                pltpu.async_copy(
                    a_hbm.at[idx2d.at[t * NBUF + b]], bufs.at[b], gsems[b]
                )
            for b in range(NBUF):
                base = wid * EPW + (t * NBUF + b) * CG
                pltpu.make_async_copy(
                    a_hbm.at[idx2d.at[0]], bufs.at[b], gsems[b]
                ).wait()
                pltpu.async_copy(bufs.at[b], g_hbm.at[pl.ds(base, CG)], wsems[b])
            return carry

        lax.fori_loop(0, NCHG // NBUF, body, 0)
        for b in range(NBUF):
            pltpu.make_async_copy(
                bufs.at[b], g_hbm.at[pl.ds(wid * EPW, CG)], wsems[b]
            ).wait()

    return k(a, src2)


def _scatter(r, dst2, zrows):
    @functools.partial(
        pl.kernel,
        out_type=jax.ShapeDtypeStruct((NC, N, H), jnp.float32),
        mesh=_sc_mesh(),
        scratch_types=[
            pltpu.VMEM((NBUF, CS), jnp.int32),
            pltpu.VMEM((NBUF, CS, H), jnp.float32),
            pltpu.VMEM_SHARED((N, H), jnp.float32),
            [pltpu.SemaphoreType.DMA] * NBUF,
            [pltpu.SemaphoreType.DMA] * NBUF,
        ],
    )
    def k(r_hbm, dst_hbm, z_hbm, out_hbm, idxb, bufs, s_sh, lsems, asems):
        cid = lax.axis_index("c")
        sid = lax.axis_index("s")
        wid = sid * NC + cid

        @pl.when(sid < NTC)
        def _zero():
            pltpu.sync_copy(z_hbm, s_sh.at[pl.ds(sid * CPT, CPT)])

        plsc.subcore_barrier()

        def body(t, carry):
            for b in range(NBUF):
                j = t * NBUF + b

                @pl.when(t > 0)
                def _drain_a(b=b):
                    pltpu.make_async_copy(
                        bufs.at[b], s_sh.at[idxb.at[b]], asems[b]
                    ).wait()

                pltpu.async_copy(dst_hbm.at[wid].at[j], idxb.at[b], lsems[b])
                base = wid * EPW + j * CS
                pltpu.async_copy(r_hbm.at[pl.ds(base, CS)], bufs.at[b], lsems[b])
            for b in range(NBUF):
                j = t * NBUF + b
                pltpu.make_async_copy(
                    dst_hbm.at[wid].at[0], idxb.at[b], lsems[b]
                ).wait()
                pltpu.make_async_copy(
                    r_hbm.at[pl.ds(wid * EPW, CS)], bufs.at[b], lsems[b]
                ).wait()
                pltpu.async_copy(
                    bufs.at[b], s_sh.at[idxb.at[b]], asems[b], add=True
                )
            return carry

        lax.fori_loop(0, NCHS // NBUF, body, 0)
        for b in range(NBUF):
            pltpu.make_async_copy(bufs.at[b], s_sh.at[idxb.at[b]], asems[b]).wait()
        plsc.subcore_barrier()

        @pl.when(sid < NTC)
        def _out():
            pltpu.sync_copy(
                s_sh.at[pl.ds(sid * CPT, CPT)],
                out_hbm.at[cid].at[pl.ds(sid * CPT, CPT)],
            )

    return k(r, dst2, zrows)


def _degree(dst2, ones_rows, zrows):
    @functools.partial(
        pl.kernel,
        out_type=jax.ShapeDtypeStruct((NC, N, H), jnp.float32),
        mesh=_sc_mesh(),
        scratch_types=[
            pltpu.VMEM((NCHS, CS), jnp.int32),
            pltpu.VMEM((CS, H), jnp.float32),
            pltpu.VMEM_SHARED((N, H), jnp.float32),
            pltpu.SemaphoreType.DMA,
        ],
    )
    def k(dst_hbm, ones_hbm, z_hbm, out_hbm, idx2d, ones_v, cnt_sh, sem):
        cid = lax.axis_index("c")
        sid = lax.axis_index("s")
        wid = sid * NC + cid
        pltpu.sync_copy(dst_hbm.at[wid], idx2d)
        pltpu.sync_copy(ones_hbm, ones_v)

        @pl.when(sid < NTC)
        def _zero():
            pltpu.sync_copy(z_hbm, cnt_sh.at[pl.ds(sid * CPT, CPT)])

        plsc.subcore_barrier()

        def body(j, carry):
            d = pltpu.async_copy(ones_v, cnt_sh.at[idx2d.at[j]], sem, add=True)

            @pl.when(j >= DEPTH)
            def _drain():
                d.wait()

            return carry

        lax.fori_loop(0, NCHS, body, 0)
        for _ in range(DEPTH):
            pltpu.make_async_copy(ones_v, cnt_sh.at[idx2d.at[0]], sem).wait()
        plsc.subcore_barrier()

        @pl.when(sid < NTC)
        def _out():
            pltpu.sync_copy(
                cnt_sh.at[pl.ds(sid * CPT, CPT)],
                out_hbm.at[cid].at[pl.ds(sid * CPT, CPT)],
            )

    return k(dst2, ones_rows, zrows)


def kernel(x, edge_index, edge_attr, Wi, bi, nW1, nb1, nW2, nb2, eW1, eb1,
           eW2, eb2, mW1, mb1, mW2, mb2, gamma, beta, Wo, bo):
    src2 = edge_index[0].reshape(NW, NCHG, CG)
    dst2 = edge_index[1].reshape(NW, NCHS, CS)

    wc_all, c_all = _prep(eW2, mW1, eb2, mb1)

    zrows = jnp.zeros((CPT, H), jnp.float32)
    ones_rows = jnp.ones((CS, H), jnp.float32)

    deg = _degree(dst2, ones_rows, zrows)
    c0 = deg[0, :, 0:1]
    c1 = deg[1, :, 0:1]

    h, xn, a = _in_node(x, Wi, bi, nW1[0], nb1[0], nW2[0], nb2[0],
                        mW1[0, :H, :])
    for i in range(L):
        g = _gather(a, src2)
        r = _msg(g, edge_attr, eW1[i], eb1[i], wc_all[i], c_all[i])
        s = _scatter(r, dst2, zrows)
        if i < L - 1:
            h, xn, a = _post_node(
                s[0], s[1], c0, c1, xn, h, mW2[i], mb2[i], gamma[i], beta[i],
                nW1[i + 1], nb1[i + 1], nW2[i + 1], nb2[i + 1],
                mW1[i + 1, :H, :])
        else:
            return _post_out(s[0], s[1], c0, c1, xn, h, mW2[i], mb2[i],
                             gamma[i], beta[i], Wo, bo)


# speedup vs baseline: 3.7332x; 1.0399x over previous
import functools

import jax
import jax.numpy as jnp
from jax import lax
from jax.experimental import pallas as pl
from jax.experimental.pallas import tpu as pltpu
from jax.experimental.pallas import tpu_sc as plsc

N = 10000
E = 320000
H = 128
L = 3
IN_DIM = 128
OUT_DIM = 128

NC, NS = 2, 16
NW = NC * NS
EPW = E // NW
CG = 80
NCHG = EPW // CG
CS = 40
NCHS = EPW // CS
CPT = 1000
NTC = N // CPT

_BN_SCALE = float(1.0 / (1.0 + 1e-5) ** 0.5)


def _sc_mesh():
    return plsc.VectorSubcoreMesh(
        core_axis_name="c", subcore_axis_name="s", num_cores=NC, num_subcores=NS
    )


def _prep_body(eW2_ref, mW1_ref, eb2_ref, mb1_ref, wc_ref, c_ref):
    mW1b = mW1_ref[0, H:, :]
    wc_ref[0] = jnp.dot(eW2_ref[0], mW1b, preferred_element_type=jnp.float32)
    c_ref[0] = (
        jnp.dot(eb2_ref[0], mW1b, preferred_element_type=jnp.float32)
        + mb1_ref[0]
    )


def _prep(eW2, mW1, eb2, mb1):
    wc, c = pl.pallas_call(
        _prep_body,
        grid=(L,),
        in_specs=[
            pl.BlockSpec((1, H, H), lambda i: (i, 0, 0)),
            pl.BlockSpec((1, 2 * H, H), lambda i: (i, 0, 0)),
            pl.BlockSpec((1, 1, H), lambda i: (i, 0, 0)),
            pl.BlockSpec((1, 1, H), lambda i: (i, 0, 0)),
        ],
        out_specs=[
            pl.BlockSpec((1, H, H), lambda i: (i, 0, 0)),
            pl.BlockSpec((1, 1, H), lambda i: (i, 0, 0)),
        ],
        out_shape=[
            jax.ShapeDtypeStruct((L, H, H), jnp.float32),
            jax.ShapeDtypeStruct((L, 1, H), jnp.float32),
        ],
    )(eW2, mW1, eb2.reshape(L, 1, H), mb1.reshape(L, 1, H))
    return wc, c.reshape(L, H)


def _node_mlp(h, w1_ref, b1_ref, w2_ref, b2_ref, wa_ref):
    t = jnp.maximum(
        jnp.dot(h, w1_ref[...], preferred_element_type=jnp.float32)
        + b1_ref[...],
        0.0,
    )
    xn = jnp.dot(t, w2_ref[...], preferred_element_type=jnp.float32) + b2_ref[...]
    a = jnp.dot(xn, wa_ref[...], preferred_element_type=jnp.float32)
    return xn, a


def _in_node_body(x_ref, wi_ref, bi_ref, w1_ref, b1_ref, w2_ref, b2_ref,
                  wa_ref, h_ref, xn_ref, a_ref):
    h = (
        jnp.dot(x_ref[...], wi_ref[...], preferred_element_type=jnp.float32)
        + bi_ref[...]
    )
    h_ref[...] = h
    xn, a = _node_mlp(h, w1_ref, b1_ref, w2_ref, b2_ref, wa_ref)
    xn_ref[...] = xn
    a_ref[...] = a


def _in_node(x, Wi, bi, w1, b1, w2, b2, wa, bm=2000):
    wspec = pl.BlockSpec((H, H), lambda i: (0, 0))
    bspec = pl.BlockSpec((1, H), lambda i: (0, 0))
    rspec = pl.BlockSpec((bm, H), lambda i: (i, 0))
    return pl.pallas_call(
        _in_node_body,
        grid=(N // bm,),
        in_specs=[
            pl.BlockSpec((bm, IN_DIM), lambda i: (i, 0)),
            pl.BlockSpec((IN_DIM, H), lambda i: (0, 0)),
            bspec, wspec, bspec, wspec, bspec, wspec,
        ],
        out_specs=[rspec, rspec, rspec],
        out_shape=[
            jax.ShapeDtypeStruct((N, H), jnp.float32),
            jax.ShapeDtypeStruct((N, H), jnp.float32),
            jax.ShapeDtypeStruct((N, H), jnp.float32),
        ],
    )(x, Wi, bi.reshape(1, H), w1, b1.reshape(1, H), w2, b2.reshape(1, H), wa)


def _post_update(s0_ref, s1_ref, s2_ref, s3_ref, c0_ref, c1_ref, xn_ref,
                 h_ref, w2_ref, b2_ref, gb_ref):
    cnt = c0_ref[...] + c1_ref[...]
    s = (s0_ref[...] + s1_ref[...]) + (s2_ref[...] + s3_ref[...])
    mean = s / jnp.maximum(cnt, 1.0)
    agg = jnp.dot(mean, w2_ref[...], preferred_element_type=jnp.float32) + b2_ref[...]
    agg = jnp.where(cnt > 0.0, agg, 0.0)
    xnew = agg + xn_ref[...]
    xnew = gb_ref[0:1, :] * xnew * _BN_SCALE + gb_ref[1:2, :]
    return h_ref[...] + jnp.maximum(xnew, 0.0)


def _post_node_body(s0_ref, s1_ref, s2_ref, s3_ref, c0_ref, c1_ref, xn_ref,
                    h_ref, w2_ref, b2_ref, gb_ref, nw1_ref, nb1_ref, nw2_ref,
                    nb2_ref, wa_ref, h_out_ref, xn_out_ref, a_out_ref):
    h = _post_update(s0_ref, s1_ref, s2_ref, s3_ref, c0_ref, c1_ref, xn_ref,
                     h_ref, w2_ref, b2_ref, gb_ref)
    h_out_ref[...] = h
    xn, a = _node_mlp(h, nw1_ref, nb1_ref, nw2_ref, nb2_ref, wa_ref)
    xn_out_ref[...] = xn
    a_out_ref[...] = a


def _post_node(s0, s1, s2, s3, c0, c1, xn, h, w2, b2, gamma, beta,
               nw1, nb1, nw2, nb2, wa, bm=2000):
    gb = jnp.stack([gamma, beta], axis=0)
    wspec = pl.BlockSpec((H, H), lambda i: (0, 0))
    bspec = pl.BlockSpec((1, H), lambda i: (0, 0))
    rspec = pl.BlockSpec((bm, H), lambda i: (i, 0))
    cspec = pl.BlockSpec((bm, 1), lambda i: (i, 0))
    return pl.pallas_call(
        _post_node_body,
        grid=(N // bm,),
        in_specs=[
            rspec, rspec, rspec, rspec, cspec, cspec, rspec, rspec,
            wspec, bspec, pl.BlockSpec((2, H), lambda i: (0, 0)),
            wspec, bspec, wspec, bspec, wspec,
        ],
        out_specs=[rspec, rspec, rspec],
        out_shape=[
            jax.ShapeDtypeStruct((N, H), jnp.float32),
            jax.ShapeDtypeStruct((N, H), jnp.float32),
            jax.ShapeDtypeStruct((N, H), jnp.float32),
        ],
    )(s0, s1, s2, s3, c0, c1, xn, h, w2, b2.reshape(1, H), gb,
      nw1, nb1.reshape(1, H), nw2, nb2.reshape(1, H), wa)


def _post_out_body(s0_ref, s1_ref, s2_ref, s3_ref, c0_ref, c1_ref, xn_ref,
                   h_ref, w2_ref, b2_ref, gb_ref, wo_ref, bo_ref, o_ref):
    h = _post_update(s0_ref, s1_ref, s2_ref, s3_ref, c0_ref, c1_ref, xn_ref,
                     h_ref, w2_ref, b2_ref, gb_ref)
    o_ref[...] = (
        jnp.dot(h, wo_ref[...], preferred_element_type=jnp.float32)
        + bo_ref[...]
    )


def _post_out(s0, s1, s2, s3, c0, c1, xn, h, w2, b2, gamma, beta, Wo, bo, bm=2000):
    gb = jnp.stack([gamma, beta], axis=0)
    bspec = pl.BlockSpec((1, H), lambda i: (0, 0))
    rspec = pl.BlockSpec((bm, H), lambda i: (i, 0))
    cspec = pl.BlockSpec((bm, 1), lambda i: (i, 0))
    return pl.pallas_call(
        _post_out_body,
        grid=(N // bm,),
        in_specs=[
            rspec, rspec, rspec, rspec, cspec, cspec, rspec, rspec,
            pl.BlockSpec((H, H), lambda i: (0, 0)), bspec,
            pl.BlockSpec((2, H), lambda i: (0, 0)),
            pl.BlockSpec((H, OUT_DIM), lambda i: (0, 0)),
            pl.BlockSpec((1, OUT_DIM), lambda i: (0, 0)),
        ],
        out_specs=pl.BlockSpec((bm, OUT_DIM), lambda i: (i, 0)),
        out_shape=jax.ShapeDtypeStruct((N, OUT_DIM), jnp.float32),
    )(s0, s1, s2, s3, c0, c1, xn, h, w2, b2.reshape(1, H), gb, Wo,
      bo.reshape(1, OUT_DIM))


def _msg_body(g_ref, ea_ref, ew1_ref, eb1_ref, wc_ref, c_ref, r_ref):
    ea = ea_ref[...]
    u = (
        ea[:, 0:1] * ew1_ref[0:1, :]
        + ea[:, 1:2] * ew1_ref[1:2, :]
        + ea[:, 2:3] * ew1_ref[2:3, :]
        + eb1_ref[...]
    )
    u = jnp.maximum(u, 0.0)
    v = jnp.dot(u, wc_ref[...], preferred_element_type=jnp.float32) + c_ref[...]
    r_ref[...] = jnp.maximum(g_ref[...] + v, 0.0)


def _msg(g, ea, ew1, eb1, wc, c, bm=4000):
    ne = g.shape[0]
    return pl.pallas_call(
        _msg_body,
        grid=(ne // bm,),
        in_specs=[
            pl.BlockSpec((bm, H), lambda i: (i, 0)),
            pl.BlockSpec((bm, 3), lambda i: (i, 0)),
            pl.BlockSpec((3, H), lambda i: (0, 0)),
            pl.BlockSpec((1, H), lambda i: (0, 0)),
            pl.BlockSpec((H, H), lambda i: (0, 0)),
            pl.BlockSpec((1, H), lambda i: (0, 0)),
        ],
        out_specs=pl.BlockSpec((bm, H), lambda i: (i, 0)),
        out_shape=jax.ShapeDtypeStruct((ne, H), jnp.float32),
    )(g, ea, ew1, eb1.reshape(1, H), wc, c.reshape(1, H))


DEPTH = 8
NBUF = 5


def _gather(a, src2):
    nw, nch, cg = src2.shape
    ne = nw * nch * cg
    epw = ne // NW

    @functools.partial(
        pl.kernel,
        out_type=jax.ShapeDtypeStruct((ne, H), jnp.float32),
        mesh=_sc_mesh(),
        scratch_types=[
            pltpu.VMEM((nch, cg), jnp.int32),
            pltpu.VMEM((NBUF, cg, H), jnp.float32),
            [pltpu.SemaphoreType.DMA] * NBUF,
            [pltpu.SemaphoreType.DMA] * NBUF,
        ],
    )
    def k(a_hbm, src_hbm, g_hbm, idx2d, bufs, gsems, wsems):
        wid = lax.axis_index("s") * NC + lax.axis_index("c")
        pltpu.sync_copy(src_hbm.at[wid], idx2d)

        def body(t, carry):
            for b in range(NBUF):
                @pl.when(t > 0)
                def _drain_w(b=b):
                    pltpu.make_async_copy(
                        bufs.at[b], g_hbm.at[pl.ds(wid * epw, cg)], wsems[b]
                    ).wait()

                pltpu.async_copy(
                    a_hbm.at[idx2d.at[t * NBUF + b]], bufs.at[b], gsems[b]
                )
            for b in range(NBUF):
                base = wid * epw + (t * NBUF + b) * cg
                pltpu.make_async_copy(
                    a_hbm.at[idx2d.at[0]], bufs.at[b], gsems[b]
                ).wait()
                pltpu.async_copy(bufs.at[b], g_hbm.at[pl.ds(base, cg)], wsems[b])
            return carry

        lax.fori_loop(0, nch // NBUF, body, 0)
        for b in range(NBUF):
            pltpu.make_async_copy(
                bufs.at[b], g_hbm.at[pl.ds(wid * epw, cg)], wsems[b]
            ).wait()

    return k(a, src2)


def _scatter(r, dst2, zrows):
    nw, nch, cs = dst2.shape
    ne = nw * nch * cs
    epw = ne // NW

    @functools.partial(
        pl.kernel,
        out_type=jax.ShapeDtypeStruct((NC, N, H), jnp.float32),
        mesh=_sc_mesh(),
        scratch_types=[
            pltpu.VMEM((NBUF, CS), jnp.int32),
            pltpu.VMEM((NBUF, CS, H), jnp.float32),
            pltpu.VMEM_SHARED((N, H), jnp.float32),
            [pltpu.SemaphoreType.DMA] * NBUF,
            [pltpu.SemaphoreType.DMA] * NBUF,
        ],
    )
    def k(r_hbm, dst_hbm, z_hbm, out_hbm, idxb, bufs, s_sh, lsems, asems):
        cid = lax.axis_index("c")
        sid = lax.axis_index("s")
        wid = sid * NC + cid

        @pl.when(sid < NTC)
        def _zero():
            pltpu.sync_copy(z_hbm, s_sh.at[pl.ds(sid * CPT, CPT)])

        plsc.subcore_barrier()

        def body(t, carry):
            for b in range(NBUF):
                j = t * NBUF + b

                @pl.when(t > 0)
                def _drain_a(b=b):
                    pltpu.make_async_copy(
                        bufs.at[b], s_sh.at[idxb.at[b]], asems[b]
                    ).wait()

                pltpu.async_copy(dst_hbm.at[wid].at[j], idxb.at[b], lsems[b])
                base = wid * epw + j * CS
                pltpu.async_copy(r_hbm.at[pl.ds(base, CS)], bufs.at[b], lsems[b])
            for b in range(NBUF):
                j = t * NBUF + b
                pltpu.make_async_copy(
                    dst_hbm.at[wid].at[0], idxb.at[b], lsems[b]
                ).wait()
                pltpu.make_async_copy(
                    r_hbm.at[pl.ds(wid * epw, CS)], bufs.at[b], lsems[b]
                ).wait()
                pltpu.async_copy(
                    bufs.at[b], s_sh.at[idxb.at[b]], asems[b], add=True
                )
            return carry

        lax.fori_loop(0, nch // NBUF, body, 0)
        for b in range(NBUF):
            pltpu.make_async_copy(bufs.at[b], s_sh.at[idxb.at[b]], asems[b]).wait()
        plsc.subcore_barrier()

        @pl.when(sid < NTC)
        def _out():
            pltpu.sync_copy(
                s_sh.at[pl.ds(sid * CPT, CPT)],
                out_hbm.at[cid].at[pl.ds(sid * CPT, CPT)],
            )

    return k(r, dst2, zrows)


def _degree(dst2, ones_rows, zrows):
    @functools.partial(
        pl.kernel,
        out_type=jax.ShapeDtypeStruct((NC, N, H), jnp.float32),
        mesh=_sc_mesh(),
        scratch_types=[
            pltpu.VMEM((NCHS, CS), jnp.int32),
            pltpu.VMEM((CS, H), jnp.float32),
            pltpu.VMEM_SHARED((N, H), jnp.float32),
            pltpu.SemaphoreType.DMA,
        ],
    )
    def k(dst_hbm, ones_hbm, z_hbm, out_hbm, idx2d, ones_v, cnt_sh, sem):
        cid = lax.axis_index("c")
        sid = lax.axis_index("s")
        wid = sid * NC + cid
        pltpu.sync_copy(dst_hbm.at[wid], idx2d)
        pltpu.sync_copy(ones_hbm, ones_v)

        @pl.when(sid < NTC)
        def _zero():
            pltpu.sync_copy(z_hbm, cnt_sh.at[pl.ds(sid * CPT, CPT)])

        plsc.subcore_barrier()

        def body(j, carry):
            d = pltpu.async_copy(ones_v, cnt_sh.at[idx2d.at[j]], sem, add=True)

            @pl.when(j >= DEPTH)
            def _drain():
                d.wait()

            return carry

        lax.fori_loop(0, NCHS, body, 0)
        for _ in range(DEPTH):
            pltpu.make_async_copy(ones_v, cnt_sh.at[idx2d.at[0]], sem).wait()
        plsc.subcore_barrier()

        @pl.when(sid < NTC)
        def _out():
            pltpu.sync_copy(
                cnt_sh.at[pl.ds(sid * CPT, CPT)],
                out_hbm.at[cid].at[pl.ds(sid * CPT, CPT)],
            )

    return k(dst2, ones_rows, zrows)


def kernel(x, edge_index, edge_attr, Wi, bi, nW1, nb1, nW2, nb2, eW1, eb1,
           eW2, eb2, mW1, mb1, mW2, mb2, gamma, beta, Wo, bo):
    HE = E // 2
    src_e = edge_index[0]
    dst_e = edge_index[1]
    src_h = [src_e[h * HE:(h + 1) * HE].reshape(NW, HE // NW // CS, CS)
             for h in range(2)]
    dst_h = [dst_e[h * HE:(h + 1) * HE].reshape(NW, HE // NW // CS, CS)
             for h in range(2)]
    ea_h = [edge_attr[:HE], edge_attr[HE:]]
    dst2 = dst_e.reshape(NW, NCHS, CS)

    wc_all, c_all = _prep(eW2, mW1, eb2, mb1)

    zrows = jnp.zeros((CPT, H), jnp.float32)
    ones_rows = jnp.ones((CS, H), jnp.float32)

    deg = _degree(dst2, ones_rows, zrows)
    c0 = deg[0, :, 0:1]
    c1 = deg[1, :, 0:1]

    h, xn, a = _in_node(x, Wi, bi, nW1[0], nb1[0], nW2[0], nb2[0],
                        mW1[0, :H, :])
    for i in range(L):
        g0 = _gather(a, src_h[0])
        g1 = _gather(a, src_h[1])
        r0 = _msg(g0, ea_h[0], eW1[i], eb1[i], wc_all[i], c_all[i])
        r1 = _msg(g1, ea_h[1], eW1[i], eb1[i], wc_all[i], c_all[i])
        sA = _scatter(r0, dst_h[0], zrows)
        sB = _scatter(r1, dst_h[1], zrows)
        if i < L - 1:
            h, xn, a = _post_node(
                sA[0], sA[1], sB[0], sB[1], c0, c1, xn, h, mW2[i], mb2[i],
                gamma[i], beta[i], nW1[i + 1], nb1[i + 1], nW2[i + 1],
                nb2[i + 1], mW1[i + 1, :H, :])
        else:
            return _post_out(sA[0], sA[1], sB[0], sB[1], c0, c1, xn, h,
                             mW2[i], mb2[i], gamma[i], beta[i], Wo, bo)


# speedup vs baseline: 3.8420x; 1.0292x over previous
import functools

import jax
import jax.numpy as jnp
from jax import lax
from jax.experimental import pallas as pl
from jax.experimental.pallas import tpu as pltpu
from jax.experimental.pallas import tpu_sc as plsc

N = 10000
E = 320000
H = 128
L = 3
IN_DIM = 128
OUT_DIM = 128

NC, NS = 2, 16
NW = NC * NS
EPW = E // NW
CG = 80
NCHG = EPW // CG
CS = 40
NCHS = EPW // CS
CPT = 1000
NTC = N // CPT

_BN_SCALE = float(1.0 / (1.0 + 1e-5) ** 0.5)


def _sc_mesh():
    return plsc.VectorSubcoreMesh(
        core_axis_name="c", subcore_axis_name="s", num_cores=NC, num_subcores=NS
    )


def _prep_body(eW2_ref, mW1_ref, eb2_ref, mb1_ref, wc_ref, c_ref):
    mW1b = mW1_ref[0, H:, :]
    wc_ref[0] = jnp.dot(eW2_ref[0], mW1b, preferred_element_type=jnp.float32)
    c_ref[0] = (
        jnp.dot(eb2_ref[0], mW1b, preferred_element_type=jnp.float32)
        + mb1_ref[0]
    )


def _prep(eW2, mW1, eb2, mb1):
    wc, c = pl.pallas_call(
        _prep_body,
        grid=(L,),
        in_specs=[
            pl.BlockSpec((1, H, H), lambda i: (i, 0, 0)),
            pl.BlockSpec((1, 2 * H, H), lambda i: (i, 0, 0)),
            pl.BlockSpec((1, 1, H), lambda i: (i, 0, 0)),
            pl.BlockSpec((1, 1, H), lambda i: (i, 0, 0)),
        ],
        out_specs=[
            pl.BlockSpec((1, H, H), lambda i: (i, 0, 0)),
            pl.BlockSpec((1, 1, H), lambda i: (i, 0, 0)),
        ],
        out_shape=[
            jax.ShapeDtypeStruct((L, H, H), jnp.float32),
            jax.ShapeDtypeStruct((L, 1, H), jnp.float32),
        ],
    )(eW2, mW1, eb2.reshape(L, 1, H), mb1.reshape(L, 1, H))
    return wc, c.reshape(L, H)


def _node_mlp(h, w1_ref, b1_ref, w2_ref, b2_ref, wa_ref):
    t = jnp.maximum(
        jnp.dot(h, w1_ref[...], preferred_element_type=jnp.float32)
        + b1_ref[...],
        0.0,
    )
    xn = jnp.dot(t, w2_ref[...], preferred_element_type=jnp.float32) + b2_ref[...]
    a = jnp.dot(xn, wa_ref[...], preferred_element_type=jnp.float32)
    return xn, a


def _in_node_body(x_ref, wi_ref, bi_ref, w1_ref, b1_ref, w2_ref, b2_ref,
                  wa_ref, h_ref, xn_ref, a_ref):
    h = (
        jnp.dot(x_ref[...], wi_ref[...], preferred_element_type=jnp.float32)
        + bi_ref[...]
    )
    h_ref[...] = h
    xn, a = _node_mlp(h, w1_ref, b1_ref, w2_ref, b2_ref, wa_ref)
    xn_ref[...] = xn
    a_ref[...] = a


def _in_node(x, Wi, bi, w1, b1, w2, b2, wa, bm=2000):
    wspec = pl.BlockSpec((H, H), lambda i: (0, 0))
    bspec = pl.BlockSpec((1, H), lambda i: (0, 0))
    rspec = pl.BlockSpec((bm, H), lambda i: (i, 0))
    return pl.pallas_call(
        _in_node_body,
        grid=(N // bm,),
        in_specs=[
            pl.BlockSpec((bm, IN_DIM), lambda i: (i, 0)),
            pl.BlockSpec((IN_DIM, H), lambda i: (0, 0)),
            bspec, wspec, bspec, wspec, bspec, wspec,
        ],
        out_specs=[rspec, rspec, rspec],
        out_shape=[
            jax.ShapeDtypeStruct((N, H), jnp.float32),
            jax.ShapeDtypeStruct((N, H), jnp.float32),
            jax.ShapeDtypeStruct((N, H), jnp.float32),
        ],
    )(x, Wi, bi.reshape(1, H), w1, b1.reshape(1, H), w2, b2.reshape(1, H), wa)


def _post_update(s0_ref, s1_ref, s2_ref, s3_ref, c0_ref, c1_ref, xn_ref,
                 h_ref, w2_ref, b2_ref, gb_ref):
    cnt = c0_ref[...] + c1_ref[...]
    s = (s0_ref[...] + s1_ref[...]) + (s2_ref[...] + s3_ref[...])
    mean = s / jnp.maximum(cnt, 1.0)
    agg = jnp.dot(mean, w2_ref[...], preferred_element_type=jnp.float32) + b2_ref[...]
    agg = jnp.where(cnt > 0.0, agg, 0.0)
    xnew = agg + xn_ref[...]
    xnew = gb_ref[0:1, :] * xnew * _BN_SCALE + gb_ref[1:2, :]
    return h_ref[...] + jnp.maximum(xnew, 0.0)


def _post_node_body(s0_ref, s1_ref, s2_ref, s3_ref, c0_ref, c1_ref, xn_ref,
                    h_ref, w2_ref, b2_ref, gb_ref, nw1_ref, nb1_ref, nw2_ref,
                    nb2_ref, wa_ref, h_out_ref, xn_out_ref, a_out_ref):
    h = _post_update(s0_ref, s1_ref, s2_ref, s3_ref, c0_ref, c1_ref, xn_ref,
                     h_ref, w2_ref, b2_ref, gb_ref)
    h_out_ref[...] = h
    xn, a = _node_mlp(h, nw1_ref, nb1_ref, nw2_ref, nb2_ref, wa_ref)
    xn_out_ref[...] = xn
    a_out_ref[...] = a


def _post_node(s0, s1, s2, s3, c0, c1, xn, h, w2, b2, gamma, beta,
               nw1, nb1, nw2, nb2, wa, bm=2000):
    gb = jnp.stack([gamma, beta], axis=0)
    wspec = pl.BlockSpec((H, H), lambda i: (0, 0))
    bspec = pl.BlockSpec((1, H), lambda i: (0, 0))
    rspec = pl.BlockSpec((bm, H), lambda i: (i, 0))
    cspec = pl.BlockSpec((bm, 1), lambda i: (i, 0))
    return pl.pallas_call(
        _post_node_body,
        grid=(N // bm,),
        in_specs=[
            rspec, rspec, rspec, rspec, cspec, cspec, rspec, rspec,
            wspec, bspec, pl.BlockSpec((2, H), lambda i: (0, 0)),
            wspec, bspec, wspec, bspec, wspec,
        ],
        out_specs=[rspec, rspec, rspec],
        out_shape=[
            jax.ShapeDtypeStruct((N, H), jnp.float32),
            jax.ShapeDtypeStruct((N, H), jnp.float32),
            jax.ShapeDtypeStruct((N, H), jnp.float32),
        ],
    )(s0, s1, s2, s3, c0, c1, xn, h, w2, b2.reshape(1, H), gb,
      nw1, nb1.reshape(1, H), nw2, nb2.reshape(1, H), wa)


def _post_out_body(s0_ref, s1_ref, s2_ref, s3_ref, c0_ref, c1_ref, xn_ref,
                   h_ref, w2_ref, b2_ref, gb_ref, wo_ref, bo_ref, o_ref):
    h = _post_update(s0_ref, s1_ref, s2_ref, s3_ref, c0_ref, c1_ref, xn_ref,
                     h_ref, w2_ref, b2_ref, gb_ref)
    o_ref[...] = (
        jnp.dot(h, wo_ref[...], preferred_element_type=jnp.float32)
        + bo_ref[...]
    )


def _post_out(s0, s1, s2, s3, c0, c1, xn, h, w2, b2, gamma, beta, Wo, bo, bm=2000):
    gb = jnp.stack([gamma, beta], axis=0)
    bspec = pl.BlockSpec((1, H), lambda i: (0, 0))
    rspec = pl.BlockSpec((bm, H), lambda i: (i, 0))
    cspec = pl.BlockSpec((bm, 1), lambda i: (i, 0))
    return pl.pallas_call(
        _post_out_body,
        grid=(N // bm,),
        in_specs=[
            rspec, rspec, rspec, rspec, cspec, cspec, rspec, rspec,
            pl.BlockSpec((H, H), lambda i: (0, 0)), bspec,
            pl.BlockSpec((2, H), lambda i: (0, 0)),
            pl.BlockSpec((H, OUT_DIM), lambda i: (0, 0)),
            pl.BlockSpec((1, OUT_DIM), lambda i: (0, 0)),
        ],
        out_specs=pl.BlockSpec((bm, OUT_DIM), lambda i: (i, 0)),
        out_shape=jax.ShapeDtypeStruct((N, OUT_DIM), jnp.float32),
    )(s0, s1, s2, s3, c0, c1, xn, h, w2, b2.reshape(1, H), gb, Wo,
      bo.reshape(1, OUT_DIM))


def _msg_body(g_ref, ea_ref, ew1_ref, eb1_ref, wc_ref, c_ref, r_ref):
    ea = ea_ref[...]
    u = (
        ea[:, 0:1] * ew1_ref[0:1, :]
        + ea[:, 1:2] * ew1_ref[1:2, :]
        + ea[:, 2:3] * ew1_ref[2:3, :]
        + eb1_ref[...]
    )
    u = jnp.maximum(u, 0.0)
    v = jnp.dot(u, wc_ref[...], preferred_element_type=jnp.float32) + c_ref[...]
    r_ref[...] = jnp.maximum(g_ref[...] + v, 0.0)


def _msg(g, ea, ew1, eb1, wc, c, bm=8000):
    ne = g.shape[0]
    return pl.pallas_call(
        _msg_body,
        grid=(ne // bm,),
        in_specs=[
            pl.BlockSpec((bm, H), lambda i: (i, 0)),
            pl.BlockSpec((bm, 3), lambda i: (i, 0)),
            pl.BlockSpec((3, H), lambda i: (0, 0)),
            pl.BlockSpec((1, H), lambda i: (0, 0)),
            pl.BlockSpec((H, H), lambda i: (0, 0)),
            pl.BlockSpec((1, H), lambda i: (0, 0)),
        ],
        out_specs=pl.BlockSpec((bm, H), lambda i: (i, 0)),
        out_shape=jax.ShapeDtypeStruct((ne, H), jnp.float32),
    )(g, ea, ew1, eb1.reshape(1, H), wc, c.reshape(1, H))


DEPTH = 8
NBUF = 5


def _gather(a, src2):
    nw, nch, cg = src2.shape
    ne = nw * nch * cg
    epw = ne // NW

    @functools.partial(
        pl.kernel,
        out_type=jax.ShapeDtypeStruct((ne, H), jnp.float32),
        mesh=_sc_mesh(),
        scratch_types=[
            pltpu.VMEM((nch, cg), jnp.int32),
            pltpu.VMEM((NBUF, cg, H), jnp.float32),
            [pltpu.SemaphoreType.DMA] * NBUF,
            [pltpu.SemaphoreType.DMA] * NBUF,
        ],
    )
    def k(a_hbm, src_hbm, g_hbm, idx2d, bufs, gsems, wsems):
        wid = lax.axis_index("s") * NC + lax.axis_index("c")
        pltpu.sync_copy(src_hbm.at[wid], idx2d)

        def body(t, carry):
            for b in range(NBUF):
                @pl.when(t > 0)
                def _drain_w(b=b):
                    pltpu.make_async_copy(
                        bufs.at[b], g_hbm.at[pl.ds(wid * epw, cg)], wsems[b]
                    ).wait()

                pltpu.async_copy(
                    a_hbm.at[idx2d.at[t * NBUF + b]], bufs.at[b], gsems[b]
                )
            for b in range(NBUF):
                base = wid * epw + (t * NBUF + b) * cg
                pltpu.make_async_copy(
                    a_hbm.at[idx2d.at[0]], bufs.at[b], gsems[b]
                ).wait()
                pltpu.async_copy(bufs.at[b], g_hbm.at[pl.ds(base, cg)], wsems[b])
            return carry

        lax.fori_loop(0, nch // NBUF, body, 0)
        for b in range(NBUF):
            pltpu.make_async_copy(
                bufs.at[b], g_hbm.at[pl.ds(wid * epw, cg)], wsems[b]
            ).wait()

    return k(a, src2)


def _scatter(r, dst2, zrows):
    nw, nch, cs = dst2.shape
    ne = nw * nch * cs
    epw = ne // NW

    @functools.partial(
        pl.kernel,
        out_type=jax.ShapeDtypeStruct((NC, N, H), jnp.float32),
        mesh=_sc_mesh(),
        scratch_types=[
            pltpu.VMEM((NBUF, CS), jnp.int32),
            pltpu.VMEM((NBUF, CS, H), jnp.float32),
            pltpu.VMEM_SHARED((N, H), jnp.float32),
            [pltpu.SemaphoreType.DMA] * NBUF,
            [pltpu.SemaphoreType.DMA] * NBUF,
        ],
    )
    def k(r_hbm, dst_hbm, z_hbm, out_hbm, idxb, bufs, s_sh, lsems, asems):
        cid = lax.axis_index("c")
        sid = lax.axis_index("s")
        wid = sid * NC + cid

        @pl.when(sid < NTC)
        def _zero():
            pltpu.sync_copy(z_hbm, s_sh.at[pl.ds(sid * CPT, CPT)])

        plsc.subcore_barrier()

        def body(t, carry):
            for b in range(NBUF):
                j = t * NBUF + b

                @pl.when(t > 0)
                def _drain_a(b=b):
                    pltpu.make_async_copy(
                        bufs.at[b], s_sh.at[idxb.at[b]], asems[b]
                    ).wait()

                pltpu.async_copy(dst_hbm.at[wid].at[j], idxb.at[b], lsems[b])
                base = wid * epw + j * CS
                pltpu.async_copy(r_hbm.at[pl.ds(base, CS)], bufs.at[b], lsems[b])
            for b in range(NBUF):
                j = t * NBUF + b
                pltpu.make_async_copy(
                    dst_hbm.at[wid].at[0], idxb.at[b], lsems[b]
                ).wait()
                pltpu.make_async_copy(
                    r_hbm.at[pl.ds(wid * epw, CS)], bufs.at[b], lsems[b]
                ).wait()
                pltpu.async_copy(
                    bufs.at[b], s_sh.at[idxb.at[b]], asems[b], add=True
                )
            return carry

        lax.fori_loop(0, nch // NBUF, body, 0)
        for b in range(NBUF):
            pltpu.make_async_copy(bufs.at[b], s_sh.at[idxb.at[b]], asems[b]).wait()
        plsc.subcore_barrier()

        @pl.when(sid < NTC)
        def _out():
            pltpu.sync_copy(
                s_sh.at[pl.ds(sid * CPT, CPT)],
                out_hbm.at[cid].at[pl.ds(sid * CPT, CPT)],
            )

    return k(r, dst2, zrows)


def _degree(dst2, ones_rows, zrows):
    @functools.partial(
        pl.kernel,
        out_type=jax.ShapeDtypeStruct((NC, N, H), jnp.float32),
        mesh=_sc_mesh(),
        scratch_types=[
            pltpu.VMEM((NBUF, CS), jnp.int32),
            pltpu.VMEM((CS, H), jnp.float32),
            pltpu.VMEM_SHARED((N, H), jnp.float32),
            [pltpu.SemaphoreType.DMA] * NBUF,
            [pltpu.SemaphoreType.DMA] * NBUF,
        ],
    )
    def k(dst_hbm, ones_hbm, z_hbm, out_hbm, idxb, ones_v, cnt_sh, lsems, asems):
        cid = lax.axis_index("c")
        sid = lax.axis_index("s")
        wid = sid * NC + cid
        pltpu.sync_copy(ones_hbm, ones_v)

        @pl.when(sid < NTC)
        def _zero():
            pltpu.sync_copy(z_hbm, cnt_sh.at[pl.ds(sid * CPT, CPT)])

        plsc.subcore_barrier()

        def body(t, carry):
            for b in range(NBUF):
                j = t * NBUF + b

                @pl.when(t > 0)
                def _drain(b=b):
                    pltpu.make_async_copy(
                        ones_v, cnt_sh.at[idxb.at[b]], asems[b]
                    ).wait()

                pltpu.async_copy(dst_hbm.at[wid].at[j], idxb.at[b], lsems[b])
            for b in range(NBUF):
                pltpu.make_async_copy(
                    dst_hbm.at[wid].at[0], idxb.at[b], lsems[b]
                ).wait()
                pltpu.async_copy(
                    ones_v, cnt_sh.at[idxb.at[b]], asems[b], add=True
                )
            return carry

        lax.fori_loop(0, NCHS // NBUF, body, 0)
        for b in range(NBUF):
            pltpu.make_async_copy(ones_v, cnt_sh.at[idxb.at[b]], asems[b]).wait()
        plsc.subcore_barrier()

        @pl.when(sid < NTC)
        def _out():
            pltpu.sync_copy(
                cnt_sh.at[pl.ds(sid * CPT, CPT)],
                out_hbm.at[cid].at[pl.ds(sid * CPT, CPT)],
            )

    return k(dst2, ones_rows, zrows)


def kernel(x, edge_index, edge_attr, Wi, bi, nW1, nb1, nW2, nb2, eW1, eb1,
           eW2, eb2, mW1, mb1, mW2, mb2, gamma, beta, Wo, bo):
    HE = E // 2
    src_e = edge_index[0]
    dst_e = edge_index[1]
    src_h = [src_e[h * HE:(h + 1) * HE].reshape(NW, HE // NW // CS, CS)
             for h in range(2)]
    dst_h = [dst_e[h * HE:(h + 1) * HE].reshape(NW, HE // NW // CS, CS)
             for h in range(2)]
    ea_h = [edge_attr[:HE], edge_attr[HE:]]
    dst2 = dst_e.reshape(NW, NCHS, CS)

    wc_all, c_all = _prep(eW2, mW1, eb2, mb1)

    zrows = jnp.zeros((CPT, H), jnp.float32)
    ones_rows = jnp.ones((CS, H), jnp.float32)

    deg = _degree(dst2, ones_rows, zrows)
    c0 = deg[0, :, 0:1]
    c1 = deg[1, :, 0:1]

    h, xn, a = _in_node(x, Wi, bi, nW1[0], nb1[0], nW2[0], nb2[0],
                        mW1[0, :H, :])
    for i in range(L):
        g0 = _gather(a, src_h[0])
        g1 = _gather(a, src_h[1])
        r0 = _msg(g0, ea_h[0], eW1[i], eb1[i], wc_all[i], c_all[i])
        r1 = _msg(g1, ea_h[1], eW1[i], eb1[i], wc_all[i], c_all[i])
        sA = _scatter(r0, dst_h[0], zrows)
        sB = _scatter(r1, dst_h[1], zrows)
        if i < L - 1:
            h, xn, a = _post_node(
                sA[0], sA[1], sB[0], sB[1], c0, c1, xn, h, mW2[i], mb2[i],
                gamma[i], beta[i], nW1[i + 1], nb1[i + 1], nW2[i + 1],
                nb2[i + 1], mW1[i + 1, :H, :])
        else:
            return _post_out(sA[0], sA[1], sB[0], sB[1], c0, c1, xn, h,
                             mW2[i], mb2[i], gamma[i], beta[i], Wo, bo)
